# baseline (speedup 1.0000x reference)
import functools

import jax
import jax.numpy as jnp
from jax import lax
from jax.experimental import pallas as pl
from jax.experimental.pallas import tpu as pltpu
from jax.experimental.pallas import tpu_sc as plsc

NUM_EMBEDDINGS = 1000000
EMBED_DIM = 64
BATCH = 16384
HIST = 50

NC = 2
NS = 16
NW = NC * NS

B = BATCH * HIST
NBLK = BATCH // 128
NUNIT = HIST * NBLK
U_PER_W = NUNIT // NW
CHUNK = 128

TBLK = 7808
T_PER_W = TBLK // NW
TAIL_ROWS = NUM_EMBEDDINGS - TBLK * 128


def _make_untile():
    mesh = plsc.VectorSubcoreMesh(core_axis_name="c", subcore_axis_name="s")

    @functools.partial(
        pl.kernel,
        mesh=mesh,
        out_type=jax.ShapeDtypeStruct((NUM_EMBEDDINGS // 2, 128), jnp.float32),
        scratch_types=[
            [pltpu.VMEM((64, 128), jnp.float32) for _ in range(4)],
            [pltpu.VMEM((64, 128), jnp.float32) for _ in range(4)],
            [pltpu.SemaphoreType.DMA for _ in range(4)],
            [pltpu.SemaphoreType.DMA for _ in range(4)],
        ],
        compiler_params=pltpu.CompilerParams(
            use_tc_tiling_on_sc=True, needs_layout_passes=False
        ),
    )
    def untile(tblt_hbm, out_hbm, ins, outs, sem_i, sem_o):
        wid = lax.axis_index("s") * NC + lax.axis_index("c")
        c0 = wid * T_PER_W

        iota16 = lax.iota(jnp.int32, 16)
        skew = [lax.bitwise_and(iota16 + k, 15) for k in range(16)]
        stconst = [iota16 * 64 + skew[k] for k in range(16)]

        def fire_read(c, b):
            pltpu.async_copy(tblt_hbm.at[:, pl.ds(c * 128, 128)], ins[b], sem_i[b])

        def wait_read(b):
            pltpu.make_async_copy(
                tblt_hbm.at[:, pl.ds(0, 128)], ins[b], sem_i[b]
            ).wait()

        def fire_write(c, b):
            pltpu.async_copy(outs[b], out_hbm.at[pl.ds(c * 64, 64)], sem_o[b])

        def wait_write(b):
            pltpu.make_async_copy(outs[b], out_hbm.at[pl.ds(0, 64)], sem_o[b]).wait()

        def transpose_block(b):
            @plsc.parallel_loop(0, 32, unroll=4)
            def blk_body(i):
                d0 = (i // 8) * 16
                t0 = (i % 8) * 16
                tvec = t0 + iota16
                for k in range(16):
                    dvec = d0 + skew[k]
                    vals = plsc.load_gather(ins[b], [dvec, tvec])
                    flat = (t0 * 64 + d0) + stconst[k]
                    ridx = lax.shift_right_logical(flat, 7)
                    cidx = lax.bitwise_and(flat, 127)
                    plsc.store_scatter(outs[b], [ridx, cidx], vals)

        for b in range(4):
            fire_read(c0 + b, b)

        def ring_body(g, carry):
            for b in range(4):
                j = g * 4 + b
                wait_read(b)

                @pl.when(j >= 4)
                def _():
                    wait_write(b)

                transpose_block(b)
                fire_write(c0 + j, b)

                @pl.when(j + 4 < T_PER_W)
                def _():
                    fire_read(c0 + j + 4, b)
            return carry

        lax.fori_loop(0, T_PER_W // 4, ring_body, 0)
        for b in range(4):
            wait_write(b)

    return untile


def _make_gather():
    mesh = plsc.VectorSubcoreMesh(core_axis_name="c", subcore_axis_name="s")

    @functools.partial(
        pl.kernel,
        mesh=mesh,
        out_type=jax.ShapeDtypeStruct((HIST, 8, NBLK, 1024), jnp.float32),
        scratch_types=[
            pltpu.VMEM((U_PER_W, CHUNK), jnp.int32),
            [pltpu.VMEM((CHUNK, 64), jnp.float32) for _ in range(4)],
            [pltpu.VMEM((8, 1024), jnp.float32) for _ in range(4)],
            [pltpu.SemaphoreType.DMA for _ in range(4)],
            [pltpu.SemaphoreType.DMA for _ in range(4)],
        ],
        compiler_params=pltpu.CompilerParams(
            use_tc_tiling_on_sc=False, needs_layout_passes=False
        ),
    )
    def emb_gather(idx_hbm, table_hbm, out_hbm, idx_v, rows, trs, sem_g, sem_w):
        wid = lax.axis_index("s") * NC + lax.axis_index("c")
        u0 = wid * U_PER_W
        pltpu.sync_copy(idx_hbm.at[wid], idx_v)

        iota16 = lax.iota(jnp.int32, 16)
        skew = [lax.bitwise_and(iota16 + k, 15) for k in range(16)]

        def fire_gather(j, b):
            pltpu.async_copy(table_hbm.at[idx_v.at[j]], rows[b], sem_g[b])

        def wait_gather(b):
            pltpu.make_async_copy(
                table_hbm.at[idx_v.at[0]], rows[b], sem_g[b]
            ).wait()

        def fire_writeback(u, b):
            h = u // NBLK
            c = u % NBLK
            pltpu.async_copy(trs[b], out_hbm.at[h, :, c], sem_w[b])

        def wait_writeback(b):
            pltpu.make_async_copy(trs[b], out_hbm.at[0, :, 0], sem_w[b]).wait()

        def transpose_unit(b):
            @plsc.parallel_loop(0, 32, unroll=4)
            def blk_body(i):
                d0 = (i // 8) * 16
                t0 = (i % 8) * 16
                trow = t0 + iota16
                for k in range(16):
                    tcol = d0 + skew[k]
                    vals = plsc.load_gather(rows[b], [trow, tcol])
                    ridx = lax.shift_right_logical(tcol, 3)
                    cidx = lax.shift_left(lax.bitwise_and(tcol, 7), 7) + trow
                    plsc.store_scatter(trs[b], [ridx, cidx], vals)

        for b in range(4):
            fire_gather(b, b)

        def ring_body(g, carry):
            for b in range(4):
                j = g * 4 + b
                wait_gather(b)

                @pl.when(j >= 4)
                def _():
                    wait_writeback(b)

                transpose_unit(b)
                fire_writeback(u0 + j, b)

                @pl.when(j + 4 < U_PER_W)
                def _():
                    fire_gather(j + 4, b)
            return carry

        lax.fori_loop(0, U_PER_W // 4, ring_body, 0)
        for b in range(4):
            wait_writeback(b)

    return emb_gather


_untile = _make_untile()
_emb_gather = _make_gather()


@jax.jit
def kernel(token_ids, lookup_table):
    tbl_packed = _untile(lookup_table.T)
    tail = lookup_table[TBLK * 128 :].reshape(TAIL_ROWS // 2, 128)
    tbl_packed = lax.dynamic_update_slice(tbl_packed, tail, (TBLK * 64, 0))
    tbl_lin = tbl_packed.reshape(NUM_EMBEDDINGS, EMBED_DIM)

    tok3 = token_ids.T.reshape(NW, U_PER_W, CHUNK)
    out5 = _emb_gather(tok3, tbl_lin)
    out = (
        out5.reshape(HIST, 8, NBLK, 8, 128)
        .transpose(2, 4, 0, 1, 3)
        .reshape(BATCH, HIST, EMBED_DIM)
    )
    return out


# speedup vs baseline: 1.0394x; 1.0394x over previous
import functools

import jax
import jax.numpy as jnp
from jax import lax
from jax.experimental import pallas as pl
from jax.experimental.pallas import tpu as pltpu
from jax.experimental.pallas import tpu_sc as plsc

NUM_EMBEDDINGS = 1000000
EMBED_DIM = 64
BATCH = 16384
HIST = 50

NC = 2
NS = 16
NW = NC * NS

B = BATCH * HIST
NBLK = BATCH // 128
NUNIT = HIST * NBLK
U_PER_W = NUNIT // NW
CHUNK = 128

TBLK = 7808
T_PER_W = TBLK // NW
TAIL_ROWS = NUM_EMBEDDINGS - TBLK * 128


def _make_untile():
    mesh = plsc.VectorSubcoreMesh(core_axis_name="c", subcore_axis_name="s")

    @functools.partial(
        pl.kernel,
        mesh=mesh,
        out_type=jax.ShapeDtypeStruct((NUM_EMBEDDINGS // 2, 128), jnp.float32),
        scratch_types=[
            [pltpu.VMEM((64, 128), jnp.float32) for _ in range(4)],
            [pltpu.VMEM((64, 128), jnp.float32) for _ in range(4)],
            [pltpu.SemaphoreType.DMA for _ in range(4)],
            [pltpu.SemaphoreType.DMA for _ in range(4)],
        ],
        compiler_params=pltpu.CompilerParams(
            use_tc_tiling_on_sc=True, needs_layout_passes=False
        ),
    )
    def untile(tblt_hbm, out_hbm, ins, outs, sem_i, sem_o):
        wid = lax.axis_index("s") * NC + lax.axis_index("c")
        c0 = wid * T_PER_W

        iota16 = lax.iota(jnp.int32, 16)
        skew = [lax.bitwise_and(iota16 + k, 15) for k in range(16)]
        stconst = [iota16 * 64 + skew[k] for k in range(16)]

        def fire_read(c, b):
            pltpu.async_copy(tblt_hbm.at[:, pl.ds(c * 128, 128)], ins[b], sem_i[b])

        def wait_read(b):
            pltpu.make_async_copy(
                tblt_hbm.at[:, pl.ds(0, 128)], ins[b], sem_i[b]
            ).wait()

        def fire_write(c, b):
            pltpu.async_copy(outs[b], out_hbm.at[pl.ds(c * 64, 64)], sem_o[b])

        def wait_write(b):
            pltpu.make_async_copy(outs[b], out_hbm.at[pl.ds(0, 64)], sem_o[b]).wait()

        def transpose_block(b):
            @plsc.parallel_loop(0, 32, unroll=2)
            def blk_body(i):
                d0 = (i // 8) * 16
                t0 = (i % 8) * 16
                tvec = t0 + iota16
                for k in range(16):
                    dvec = d0 + skew[k]
                    vals = plsc.load_gather(ins[b], [dvec, tvec])
                    flat = (t0 * 64 + d0) + stconst[k]
                    ridx = lax.shift_right_logical(flat, 7)
                    cidx = lax.bitwise_and(flat, 127)
                    plsc.store_scatter(outs[b], [ridx, cidx], vals)

        for b in range(4):
            fire_read(c0 + b, b)

        def ring_body(g, carry):
            for b in range(4):
                j = g * 4 + b
                wait_read(b)

                @pl.when(j >= 4)
                def _():
                    wait_write(b)

                transpose_block(b)
                fire_write(c0 + j, b)

                @pl.when(j + 4 < T_PER_W)
                def _():
                    fire_read(c0 + j + 4, b)
            return carry

        lax.fori_loop(0, T_PER_W // 4, ring_body, 0)
        for b in range(4):
            wait_write(b)

    return untile


def _make_gather():
    mesh = plsc.VectorSubcoreMesh(core_axis_name="c", subcore_axis_name="s")

    @functools.partial(
        pl.kernel,
        mesh=mesh,
        out_type=jax.ShapeDtypeStruct((HIST, 8, NBLK, 1024), jnp.float32),
        scratch_types=[
            pltpu.VMEM((U_PER_W, CHUNK), jnp.int32),
            [pltpu.VMEM((CHUNK, 64), jnp.float32) for _ in range(4)],
            [pltpu.VMEM((8, 1024), jnp.float32) for _ in range(4)],
            [pltpu.SemaphoreType.DMA for _ in range(4)],
            [pltpu.SemaphoreType.DMA for _ in range(4)],
        ],
        compiler_params=pltpu.CompilerParams(
            use_tc_tiling_on_sc=False, needs_layout_passes=False
        ),
    )
    def emb_gather(idx_hbm, table_hbm, out_hbm, idx_v, rows, trs, sem_g, sem_w):
        wid = lax.axis_index("s") * NC + lax.axis_index("c")
        u0 = wid * U_PER_W
        pltpu.sync_copy(idx_hbm.at[wid], idx_v)

        iota16 = lax.iota(jnp.int32, 16)
        skew = [lax.bitwise_and(iota16 + k, 15) for k in range(16)]

        def fire_gather(j, b):
            pltpu.async_copy(table_hbm.at[idx_v.at[j]], rows[b], sem_g[b])

        def wait_gather(b):
            pltpu.make_async_copy(
                table_hbm.at[idx_v.at[0]], rows[b], sem_g[b]
            ).wait()

        def fire_writeback(u, b):
            h = u // NBLK
            c = u % NBLK
            pltpu.async_copy(trs[b], out_hbm.at[h, :, c], sem_w[b])

        def wait_writeback(b):
            pltpu.make_async_copy(trs[b], out_hbm.at[0, :, 0], sem_w[b]).wait()

        def transpose_unit(b):
            @plsc.parallel_loop(0, 32, unroll=2)
            def blk_body(i):
                d0 = (i // 8) * 16
                t0 = (i % 8) * 16
                trow = t0 + iota16
                for k in range(16):
                    tcol = d0 + skew[k]
                    vals = plsc.load_gather(rows[b], [trow, tcol])
                    ridx = lax.shift_right_logical(tcol, 3)
                    cidx = lax.shift_left(lax.bitwise_and(tcol, 7), 7) + trow
                    plsc.store_scatter(trs[b], [ridx, cidx], vals)

        for b in range(4):
            fire_gather(b, b)

        def ring_body(g, carry):
            for b in range(4):
                j = g * 4 + b
                wait_gather(b)

                @pl.when(j >= 4)
                def _():
                    wait_writeback(b)

                transpose_unit(b)
                fire_writeback(u0 + j, b)

                @pl.when(j + 4 < U_PER_W)
                def _():
                    fire_gather(j + 4, b)
            return carry

        lax.fori_loop(0, U_PER_W // 4, ring_body, 0)
        for b in range(4):
            wait_writeback(b)

    return emb_gather


_untile = _make_untile()
_emb_gather = _make_gather()


@jax.jit
def kernel(token_ids, lookup_table):
    tbl_packed = _untile(lookup_table.T)
    tail = lookup_table[TBLK * 128 :].reshape(TAIL_ROWS // 2, 128)
    tbl_packed = lax.dynamic_update_slice(tbl_packed, tail, (TBLK * 64, 0))
    tbl_lin = tbl_packed.reshape(NUM_EMBEDDINGS, EMBED_DIM)

    tok3 = token_ids.T.reshape(NW, U_PER_W, CHUNK)
    out5 = _emb_gather(tok3, tbl_lin)
    out = (
        out5.reshape(HIST, 8, NBLK, 8, 128)
        .transpose(2, 4, 0, 1, 3)
        .reshape(BATCH, HIST, EMBED_DIM)
    )
    return out
